# baseline (speedup 1.0000x reference)
import jax
import jax.numpy as jnp
from jax.experimental import pallas as pl
from jax.experimental.pallas import tpu as pltpu


def _cheb_body(mid_lo, mid_hi):
    def body(xr_ref, w0t_ref, w1t_ref, bbt_ref, out_ref, a0_ref, a1_ref):
        for i, a_ref in ((0, a0_ref), (1, a1_ref)):
            xb = xr_ref[i]
            g = xb[:, mid_lo:mid_hi] * jnp.float32(1.4142135623730951)
            sq = 0.5 * jnp.sum(g * g, axis=1, keepdims=True)
            gg = jax.lax.dot_general(
                g, g, (((1,), (1,)), ((), ())),
                preferred_element_type=jnp.float32)
            arg = gg - sq - jnp.transpose(sq)
            a_ref[...] = jnp.exp(jnp.minimum(arg, 0.0)).astype(jnp.bfloat16)
        for i, a_ref in ((0, a0_ref), (1, a1_ref)):
            xb = xr_ref[i]
            deg = jnp.sum(a_ref[...].astype(jnp.float32), axis=1,
                          keepdims=True)
            dinv = jax.lax.rsqrt(deg + 1e-6)
            y = (dinv * xb).astype(jnp.bfloat16)
            tx1 = (dinv * jnp.dot(a_ref[...], y,
                                  preferred_element_type=jnp.float32)
                   ).astype(jnp.bfloat16)
            out = (jax.lax.dot_general(w0t_ref[...], xb.astype(jnp.bfloat16),
                                       (((1,), (1,)), ((), ())),
                                       preferred_element_type=jnp.float32)
                   + jax.lax.dot_general(w1t_ref[...], tx1,
                                         (((1,), (1,)), ((), ())),
                                         preferred_element_type=jnp.float32)
                   + bbt_ref[...][:, :1])
            out_ref[i] = jnp.maximum(out, 0.0).astype(jnp.bfloat16)
    return body


def _gru_body(n_k, n_t, n_b, n_o, hid):
    def body(ot_ref, w5_ref, bih_ref, whh_ref, bhh_ref, wfc_ref, bfc_ref,
             out_ref, acc_ref):
        k = pl.program_id(0)

        @pl.when(k == 0)
        def _init():
            acc_ref[...] = jnp.zeros_like(acc_ref)

        ot = ot_ref[...]
        w3 = w5_ref[...].astype(jnp.bfloat16)
        kn = ot.shape[-1]
        part = jnp.zeros((n_b * n_t, 3 * hid), dtype=jnp.float32)
        for o in range(n_o):
            lhs = ot[:, o * n_t:(o + 1) * n_t, :].reshape(n_b * n_t, kn)
            rhs = w3[:, o, :]
            part = part + jnp.dot(lhs, rhs,
                                  preferred_element_type=jnp.float32)
        acc_ref[...] += part

        @pl.when(k == n_k - 1)
        def _finish():
            gx = acc_ref[...] + bih_ref[...]
            whh = whh_ref[...]
            bhh = bhh_ref[...]
            gx3 = gx.reshape(n_b, n_t, 3 * hid)
            h = jnp.zeros((n_b, hid), dtype=jnp.float32)
            for t in range(n_t):
                gxt = gx3[:, t, :]
                gh = jnp.dot(h, whh,
                             preferred_element_type=jnp.float32) + bhh
                r = jax.nn.sigmoid(gxt[:, :hid] + gh[:, :hid])
                z = jax.nn.sigmoid(gxt[:, hid:2 * hid] + gh[:, hid:2 * hid])
                n = jnp.tanh(gxt[:, 2 * hid:] + r * gh[:, 2 * hid:])
                h = (1.0 - z) * n + z * h
            out_ref[...] = jnp.dot(h, wfc_ref[...],
                                   preferred_element_type=jnp.float32) \
                + bfc_ref[...]
    return body


def kernel(x, W_cheb, b_cheb, W_ih, W_hh, b_ih, b_hh, W_fc, b_fc):
    B, T, N, F = x.shape
    O = W_cheb.shape[-1]
    TF = T * F
    TO = T * O
    HID = W_hh.shape[0]
    HOUT = W_fc.shape[-1]
    mid = T // 2

    xr = x.transpose(0, 2, 1, 3).reshape(B, N, TF)
    eyeT = jnp.eye(T, dtype=x.dtype)
    idx = jnp.arange(TO)
    perm = (idx % T) * O + idx // T
    w0t = jnp.kron(eyeT, W_cheb[0]).T[perm].astype(jnp.bfloat16)
    w1t = jnp.kron(eyeT, W_cheb[1]).T[perm].astype(jnp.bfloat16)
    bbt = jnp.broadcast_to(jnp.repeat(b_cheb, T)[:, None], (TO, 128))

    cheb = pl.pallas_call(
        _cheb_body(mid * F, (mid + 1) * F),
        grid=(B // 2,),
        in_specs=[
            pl.BlockSpec((2, N, TF), lambda b: (b, 0, 0)),
            pl.BlockSpec((TO, TF), lambda b: (0, 0)),
            pl.BlockSpec((TO, TF), lambda b: (0, 0)),
            pl.BlockSpec((TO, 128), lambda b: (0, 0)),
        ],
        out_specs=pl.BlockSpec((2, TO, N), lambda b: (b, 0, 0)),
        out_shape=jax.ShapeDtypeStruct((B, TO, N), jnp.bfloat16),
        scratch_shapes=[pltpu.VMEM((N, N), jnp.bfloat16),
                        pltpu.VMEM((N, N), jnp.bfloat16)],
    )
    OT = cheb(xr, w0t, w1t, bbt)

    W4 = W_ih.reshape(N, O, 3 * HID)
    KN = 256
    n_k = N // KN
    gru = pl.pallas_call(
        _gru_body(n_k, T, B, O, HID),
        grid=(n_k,),
        in_specs=[
            pl.BlockSpec((B, TO, KN), lambda k: (0, 0, k)),
            pl.BlockSpec((KN, O, 3 * HID), lambda k: (k, 0, 0)),
            pl.BlockSpec((1, 3 * HID), lambda k: (0, 0)),
            pl.BlockSpec((HID, 3 * HID), lambda k: (0, 0)),
            pl.BlockSpec((1, 3 * HID), lambda k: (0, 0)),
            pl.BlockSpec((HID, HOUT), lambda k: (0, 0)),
            pl.BlockSpec((1, HOUT), lambda k: (0, 0)),
        ],
        out_specs=pl.BlockSpec((B, HOUT), lambda k: (0, 0)),
        out_shape=jax.ShapeDtypeStruct((B, HOUT), jnp.float32),
        scratch_shapes=[pltpu.VMEM((T * B, 3 * HID), jnp.float32)],
    )
    return gru(OT, W4, b_ih[None, :], W_hh, b_hh[None, :],
               W_fc, b_fc[None, :])


# speedup vs baseline: 1.0563x; 1.0563x over previous
import jax
import jax.numpy as jnp
from jax.experimental import pallas as pl
from jax.experimental.pallas import tpu as pltpu


def _cheb_body(mid_lo, mid_hi):
    def body(xr_ref, w0t_ref, w1t_ref, bbt_ref, out_ref, a_ref):
        xb = xr_ref[0]
        g = xb[:, mid_lo:mid_hi] * jnp.float32(1.6986436005760381)
        sq = 0.5 * jnp.sum(g * g, axis=1, keepdims=True)
        gg = jax.lax.dot_general(
            g, g, (((1,), (1,)), ((), ())),
            preferred_element_type=jnp.float32)
        arg = gg - sq - jnp.transpose(sq)
        a = jnp.exp2(jnp.minimum(arg, 0.0))
        a_ref[...] = a.astype(jnp.bfloat16)
        deg = jnp.sum(a, axis=1, keepdims=True)
        dinv = jax.lax.rsqrt(deg + 1e-6)
        y = (dinv * xb).astype(jnp.bfloat16)
        tx1 = (dinv * jnp.dot(a_ref[...], y,
                              preferred_element_type=jnp.float32)
               ).astype(jnp.bfloat16)
        out = (jax.lax.dot_general(w0t_ref[...], xb.astype(jnp.bfloat16),
                                   (((1,), (1,)), ((), ())),
                                   preferred_element_type=jnp.float32)
               + jax.lax.dot_general(w1t_ref[...], tx1,
                                     (((1,), (1,)), ((), ())),
                                     preferred_element_type=jnp.float32)
               + bbt_ref[...][:, :1])
        out_ref[0] = jnp.maximum(out, 0.0).astype(jnp.bfloat16)
    return body


def _gru_body(n_k, n_t, n_b, n_o, hid):
    def body(ot_ref, w5_ref, bih_ref, whh_ref, bhh_ref, wfc_ref, bfc_ref,
             out_ref, acc_ref):
        k = pl.program_id(0)

        @pl.when(k == 0)
        def _init():
            acc_ref[...] = jnp.zeros_like(acc_ref)

        ot = ot_ref[...]
        w3 = w5_ref[...].astype(jnp.bfloat16)
        kn = ot.shape[-1]
        part = jnp.zeros((n_b * n_t, 3 * hid), dtype=jnp.float32)
        for o in range(n_o):
            lhs = ot[:, o * n_t:(o + 1) * n_t, :].reshape(n_b * n_t, kn)
            rhs = w3[:, o, :]
            part = part + jnp.dot(lhs, rhs,
                                  preferred_element_type=jnp.float32)
        acc_ref[...] += part

        @pl.when(k == n_k - 1)
        def _finish():
            gx = acc_ref[...] + bih_ref[...]
            whh = whh_ref[...]
            bhh = bhh_ref[...]
            gx3 = gx.reshape(n_b, n_t, 3 * hid)
            h = jnp.zeros((n_b, hid), dtype=jnp.float32)
            for t in range(n_t):
                gxt = gx3[:, t, :]
                gh = jnp.dot(h, whh,
                             preferred_element_type=jnp.float32) + bhh
                r = jax.nn.sigmoid(gxt[:, :hid] + gh[:, :hid])
                z = jax.nn.sigmoid(gxt[:, hid:2 * hid] + gh[:, hid:2 * hid])
                n = jnp.tanh(gxt[:, 2 * hid:] + r * gh[:, 2 * hid:])
                h = (1.0 - z) * n + z * h
            out_ref[...] = jnp.dot(h, wfc_ref[...],
                                   preferred_element_type=jnp.float32) \
                + bfc_ref[...]
    return body


def kernel(x, W_cheb, b_cheb, W_ih, W_hh, b_ih, b_hh, W_fc, b_fc):
    B, T, N, F = x.shape
    O = W_cheb.shape[-1]
    TF = T * F
    TO = T * O
    HID = W_hh.shape[0]
    HOUT = W_fc.shape[-1]
    mid = T // 2

    xr = x.transpose(0, 2, 1, 3).reshape(B, N, TF)
    eyeT = jnp.eye(T, dtype=x.dtype)
    idx = jnp.arange(TO)
    perm = (idx % T) * O + idx // T
    w0t = jnp.kron(eyeT, W_cheb[0]).T[perm].astype(jnp.bfloat16)
    w1t = jnp.kron(eyeT, W_cheb[1]).T[perm].astype(jnp.bfloat16)
    bbt = jnp.broadcast_to(jnp.repeat(b_cheb, T)[:, None], (TO, 128))

    cheb = pl.pallas_call(
        _cheb_body(mid * F, (mid + 1) * F),
        grid=(B,),
        in_specs=[
            pl.BlockSpec((1, N, TF), lambda b: (b, 0, 0)),
            pl.BlockSpec((TO, TF), lambda b: (0, 0)),
            pl.BlockSpec((TO, TF), lambda b: (0, 0)),
            pl.BlockSpec((TO, 128), lambda b: (0, 0)),
        ],
        out_specs=pl.BlockSpec((1, TO, N), lambda b: (b, 0, 0)),
        out_shape=jax.ShapeDtypeStruct((B, TO, N), jnp.bfloat16),
        scratch_shapes=[pltpu.VMEM((N, N), jnp.bfloat16)],
    )
    OT = cheb(xr, w0t, w1t, bbt)

    W4 = W_ih.reshape(N, O, 3 * HID)
    KN = 512
    n_k = N // KN
    gru = pl.pallas_call(
        _gru_body(n_k, T, B, O, HID),
        grid=(n_k,),
        in_specs=[
            pl.BlockSpec((B, TO, KN), lambda k: (0, 0, k)),
            pl.BlockSpec((KN, O, 3 * HID), lambda k: (k, 0, 0)),
            pl.BlockSpec((1, 3 * HID), lambda k: (0, 0)),
            pl.BlockSpec((HID, 3 * HID), lambda k: (0, 0)),
            pl.BlockSpec((1, 3 * HID), lambda k: (0, 0)),
            pl.BlockSpec((HID, HOUT), lambda k: (0, 0)),
            pl.BlockSpec((1, HOUT), lambda k: (0, 0)),
        ],
        out_specs=pl.BlockSpec((B, HOUT), lambda k: (0, 0)),
        out_shape=jax.ShapeDtypeStruct((B, HOUT), jnp.float32),
        scratch_shapes=[pltpu.VMEM((T * B, 3 * HID), jnp.float32)],
    )
    return gru(OT, W4, b_ih[None, :], W_hh, b_hh[None, :],
               W_fc, b_fc[None, :])


# speedup vs baseline: 1.0600x; 1.0035x over previous
import jax
import jax.numpy as jnp
from jax.experimental import pallas as pl
from jax.experimental.pallas import tpu as pltpu


def _cheb_body(mid_lo, mid_hi):
    def body(xr_ref, w0t_ref, w1t_ref, bbt_ref, out_ref, a_ref):
        xb = xr_ref[0]
        g = xb[:, mid_lo:mid_hi] * jnp.float32(1.4142135623730951)
        sq = 0.5 * jnp.sum(g * g, axis=1, keepdims=True)
        gg = jax.lax.dot_general(
            g, g, (((1,), (1,)), ((), ())),
            preferred_element_type=jnp.float32)
        arg = gg - sq - jnp.transpose(sq)
        a = jnp.exp(jnp.minimum(arg, 0.0))
        a_ref[...] = a.astype(jnp.bfloat16)
        deg = jnp.sum(a, axis=1, keepdims=True)
        dinv = jax.lax.rsqrt(deg + 1e-6)
        y = (dinv * xb).astype(jnp.bfloat16)
        tx1 = (dinv * jnp.dot(a_ref[...], y,
                              preferred_element_type=jnp.float32)
               ).astype(jnp.bfloat16)
        out = (jax.lax.dot_general(w0t_ref[...], xb.astype(jnp.bfloat16),
                                   (((1,), (1,)), ((), ())),
                                   preferred_element_type=jnp.float32)
               + jax.lax.dot_general(w1t_ref[...], tx1,
                                     (((1,), (1,)), ((), ())),
                                     preferred_element_type=jnp.float32)
               + bbt_ref[...][:, :1])
        out_ref[0] = jnp.maximum(out, 0.0).astype(jnp.bfloat16)
    return body


def _gru_body(n_k, n_t, n_b, n_o, hid):
    def body(ot_ref, w5_ref, bih_ref, whh_ref, bhh_ref, wfc_ref, bfc_ref,
             out_ref, acc_ref):
        k = pl.program_id(0)

        @pl.when(k == 0)
        def _init():
            acc_ref[...] = jnp.zeros_like(acc_ref)

        ot = ot_ref[...]
        w3 = w5_ref[...].astype(jnp.bfloat16)
        kn = ot.shape[-1]
        part = jnp.zeros((n_b * n_t, 3 * hid), dtype=jnp.float32)
        for o in range(n_o):
            lhs = ot[:, o * n_t:(o + 1) * n_t, :].reshape(n_b * n_t, kn)
            rhs = w3[:, o, :]
            part = part + jnp.dot(lhs, rhs,
                                  preferred_element_type=jnp.float32)
        acc_ref[...] += part

        @pl.when(k == n_k - 1)
        def _finish():
            gx = acc_ref[...] + bih_ref[...]
            whh = whh_ref[...]
            bhh = bhh_ref[...]
            gx3 = gx.reshape(n_b, n_t, 3 * hid)
            h = jnp.zeros((n_b, hid), dtype=jnp.float32)
            for t in range(n_t):
                gxt = gx3[:, t, :]
                gh = jnp.dot(h, whh,
                             preferred_element_type=jnp.float32) + bhh
                r = jax.nn.sigmoid(gxt[:, :hid] + gh[:, :hid])
                z = jax.nn.sigmoid(gxt[:, hid:2 * hid] + gh[:, hid:2 * hid])
                n = jnp.tanh(gxt[:, 2 * hid:] + r * gh[:, 2 * hid:])
                h = (1.0 - z) * n + z * h
            out_ref[...] = jnp.dot(h, wfc_ref[...],
                                   preferred_element_type=jnp.float32) \
                + bfc_ref[...]
    return body


def kernel(x, W_cheb, b_cheb, W_ih, W_hh, b_ih, b_hh, W_fc, b_fc):
    B, T, N, F = x.shape
    O = W_cheb.shape[-1]
    TF = T * F
    TO = T * O
    HID = W_hh.shape[0]
    HOUT = W_fc.shape[-1]
    mid = T // 2

    xr = x.transpose(0, 2, 1, 3).reshape(B, N, TF)
    eyeT = jnp.eye(T, dtype=x.dtype)
    idx = jnp.arange(TO)
    perm = (idx % T) * O + idx // T
    w0t = jnp.kron(eyeT, W_cheb[0]).T[perm].astype(jnp.bfloat16)
    w1t = jnp.kron(eyeT, W_cheb[1]).T[perm].astype(jnp.bfloat16)
    bbt = jnp.broadcast_to(jnp.repeat(b_cheb, T)[:, None], (TO, 128))

    cheb = pl.pallas_call(
        _cheb_body(mid * F, (mid + 1) * F),
        grid=(B,),
        in_specs=[
            pl.BlockSpec((1, N, TF), lambda b: (b, 0, 0)),
            pl.BlockSpec((TO, TF), lambda b: (0, 0)),
            pl.BlockSpec((TO, TF), lambda b: (0, 0)),
            pl.BlockSpec((TO, 128), lambda b: (0, 0)),
        ],
        out_specs=pl.BlockSpec((1, TO, N), lambda b: (b, 0, 0)),
        out_shape=jax.ShapeDtypeStruct((B, TO, N), jnp.bfloat16),
        scratch_shapes=[pltpu.VMEM((N, N), jnp.bfloat16)],
    )
    OT = cheb(xr, w0t, w1t, bbt)

    W4 = W_ih.reshape(N, O, 3 * HID)
    KN = 256
    n_k = N // KN
    gru = pl.pallas_call(
        _gru_body(n_k, T, B, O, HID),
        grid=(n_k,),
        in_specs=[
            pl.BlockSpec((B, TO, KN), lambda k: (0, 0, k)),
            pl.BlockSpec((KN, O, 3 * HID), lambda k: (k, 0, 0)),
            pl.BlockSpec((1, 3 * HID), lambda k: (0, 0)),
            pl.BlockSpec((HID, 3 * HID), lambda k: (0, 0)),
            pl.BlockSpec((1, 3 * HID), lambda k: (0, 0)),
            pl.BlockSpec((HID, HOUT), lambda k: (0, 0)),
            pl.BlockSpec((1, HOUT), lambda k: (0, 0)),
        ],
        out_specs=pl.BlockSpec((B, HOUT), lambda k: (0, 0)),
        out_shape=jax.ShapeDtypeStruct((B, HOUT), jnp.float32),
        scratch_shapes=[pltpu.VMEM((T * B, 3 * HID), jnp.float32)],
    )
    return gru(OT, W4, b_ih[None, :], W_hh, b_hh[None, :],
               W_fc, b_fc[None, :])
